# Initial kernel scaffold; baseline (speedup 1.0000x reference)
import jax
import jax.numpy as jnp
from jax.experimental import pallas as pl


def kernel(x_text_feat, edge_index, W1, b1, W2, b2):
    raise NotImplementedError("write your pallas kernel here")


# speedup vs baseline: 20.6594x; 20.6594x over previous
import functools

import jax
import jax.numpy as jnp
from jax import lax
from jax.experimental import pallas as pl
from jax.experimental.pallas import tpu as pltpu
from jax.experimental.pallas import tpu_sc as plsc

N = 50000
E = 800000
D_IN = 768
D_HID = 64
NCLS = 7

NP = 48
N2 = N + NP
EP = 819200
G = EP // 128
HALF = 32
D2 = 16

NSC = 2
NT = 16
NW = NSC * NT

ROWS_T = N2 // NT

TG1 = G // NT
SCH1 = 40
NSC1 = TG1 // SCH1

TG2 = G // NW
SCH2 = 40
NSC2 = TG2 // SCH2

EPTD = EP // NW
NGD = EPTD // 16

RB = 512
GRID = (N + RB - 1) // RB

_mesh = plsc.VectorSubcoreMesh(core_axis_name="c", subcore_axis_name="s")
_SC_PARAMS = pltpu.CompilerParams(needs_layout_passes=False,
                                  use_tc_tiling_on_sc=False)


@functools.partial(
    pl.kernel,
    out_type=jax.ShapeDtypeStruct((NW * N2,), jnp.float32),
    mesh=_mesh,
    scratch_types=[
        pltpu.VMEM((EPTD,), jnp.int32),
        pltpu.VMEM((N2,), jnp.float32),
    ],
    compiler_params=_SC_PARAMS,
)
def _deg_kernel(dst_hbm, zeros_hbm, hist_hbm, dst_v, hist_v):
    cid = lax.axis_index("c")
    sid = lax.axis_index("s")
    wid = sid * NSC + cid
    pltpu.sync_copy(zeros_hbm, hist_v)
    pltpu.sync_copy(dst_hbm.at[pl.ds(wid * EPTD, EPTD)], dst_v)
    ones16 = jnp.ones((16,), jnp.float32)

    def body(i, carry):
        idx = dst_v[pl.ds(i * 16, 16)]
        plsc.addupdate_scatter(hist_v, [idx], ones16)
        return carry

    lax.fori_loop(0, NGD, body, 0)
    pltpu.sync_copy(hist_v, hist_hbm.at[pl.ds(wid * N2, N2)])


@functools.partial(
    pl.kernel,
    out_type=(
        jax.ShapeDtypeStruct((N2, HALF), jnp.float32),
        jax.ShapeDtypeStruct((N2, HALF), jnp.float32),
    ),
    mesh=_mesh,
    scratch_types=[
        pltpu.VMEM((SCH1, 128), jnp.int32),
        pltpu.VMEM((SCH1, 128), jnp.int32),
        pltpu.VMEM((128, HALF), jnp.float32),
        pltpu.VMEM_SHARED((N2, HALF), jnp.float32),
    ],
    compiler_params=_SC_PARAMS,
)
def _scat1_kernel(src_hbm, dst_hbm, p0_hbm, p1_hbm, z_hbm,
                  o0_hbm, o1_hbm, src_v, dst_v, rows_v, acc_s):
    cid = lax.axis_index("c")
    sid = lax.axis_index("s")
    r0 = sid * ROWS_T
    pltpu.sync_copy(z_hbm, acc_s.at[pl.ds(r0, ROWS_T)])
    plsc.subcore_barrier()

    def superchunk(t, carry):
        g0 = sid * TG1 + t * SCH1
        pltpu.sync_copy(src_hbm.at[pl.ds(g0, SCH1)], src_v)
        pltpu.sync_copy(dst_hbm.at[pl.ds(g0, SCH1)], dst_v)

        def group(j, c2):
            @pl.when(cid == 0)
            def _():
                pltpu.sync_copy(p0_hbm.at[src_v.at[j]], rows_v)

            @pl.when(cid == 1)
            def _():
                pltpu.sync_copy(p1_hbm.at[src_v.at[j]], rows_v)

            pltpu.sync_copy(rows_v, acc_s.at[dst_v.at[j]], add=True)
            return c2

        lax.fori_loop(0, SCH1, group, 0)
        return carry

    lax.fori_loop(0, NSC1, superchunk, 0)
    plsc.subcore_barrier()

    @pl.when(cid == 0)
    def _():
        pltpu.sync_copy(acc_s.at[pl.ds(r0, ROWS_T)], o0_hbm.at[pl.ds(r0, ROWS_T)])

    @pl.when(cid == 1)
    def _():
        pltpu.sync_copy(acc_s.at[pl.ds(r0, ROWS_T)], o1_hbm.at[pl.ds(r0, ROWS_T)])


@functools.partial(
    pl.kernel,
    out_type=(
        jax.ShapeDtypeStruct((N2, D2), jnp.float32),
        jax.ShapeDtypeStruct((N2, D2), jnp.float32),
    ),
    mesh=_mesh,
    scratch_types=[
        pltpu.VMEM((SCH2, 128), jnp.int32),
        pltpu.VMEM((SCH2, 128), jnp.int32),
        pltpu.VMEM((128, D2), jnp.float32),
        pltpu.VMEM_SHARED((N2, D2), jnp.float32),
    ],
    compiler_params=_SC_PARAMS,
)
def _scat2_kernel(src_hbm, dst_hbm, p2_hbm, z_hbm,
                  q0_hbm, q1_hbm, src_v, dst_v, rows_v, acc_s):
    cid = lax.axis_index("c")
    sid = lax.axis_index("s")
    wid = sid * NSC + cid
    r0 = sid * ROWS_T
    pltpu.sync_copy(z_hbm, acc_s.at[pl.ds(r0, ROWS_T)])
    plsc.subcore_barrier()

    def superchunk(t, carry):
        g0 = wid * TG2 + t * SCH2
        pltpu.sync_copy(src_hbm.at[pl.ds(g0, SCH2)], src_v)
        pltpu.sync_copy(dst_hbm.at[pl.ds(g0, SCH2)], dst_v)

        def group(j, c2):
            pltpu.sync_copy(p2_hbm.at[src_v.at[j]], rows_v)
            pltpu.sync_copy(rows_v, acc_s.at[dst_v.at[j]], add=True)
            return c2

        lax.fori_loop(0, SCH2, group, 0)
        return carry

    lax.fori_loop(0, NSC2, superchunk, 0)
    plsc.subcore_barrier()

    @pl.when(cid == 0)
    def _():
        pltpu.sync_copy(acc_s.at[pl.ds(r0, ROWS_T)], q0_hbm.at[pl.ds(r0, ROWS_T)])

    @pl.when(cid == 1)
    def _():
        pltpu.sync_copy(acc_s.at[pl.ds(r0, ROWS_T)], q1_hbm.at[pl.ds(r0, ROWS_T)])


def _mm_body(x_ref, w_ref, o_ref):
    o_ref[...] = jnp.dot(x_ref[...], w_ref[...],
                         preferred_element_type=jnp.float32)


def _scale_body(h_ref, hist_ref, p0_ref, p1_ref, dinv_ref):
    deg = 1.0 + jnp.sum(hist_ref[...], axis=0)
    dinv = lax.rsqrt(deg)[:, None]
    p = h_ref[...] * dinv
    p0_ref[...] = p[:, :HALF]
    p1_ref[...] = p[:, HALF:]
    dinv_ref[...] = dinv


def _fuse2_body(a0_ref, a1_ref, p0_ref, p1_ref, dinv_ref, b1_ref, w2_ref,
                p2_ref):
    dinv = dinv_ref[...]
    x0 = jnp.maximum((a0_ref[...] + p0_ref[...]) * dinv
                     + b1_ref[0:1, :HALF], 0.0)
    x1 = jnp.maximum((a1_ref[...] + p1_ref[...]) * dinv
                     + b1_ref[0:1, HALF:], 0.0)
    h2 = (jnp.dot(x0, w2_ref[:HALF, :], preferred_element_type=jnp.float32)
          + jnp.dot(x1, w2_ref[HALF:, :], preferred_element_type=jnp.float32))
    p2_ref[...] = h2 * dinv


def _fuse3_body(q0_ref, q1_ref, p2_ref, dinv_ref, b2_ref, o_ref):
    logits = ((q0_ref[...] + q1_ref[...] + p2_ref[...]) * dinv_ref[...]
              + b2_ref[0:1, :])
    col = lax.broadcasted_iota(jnp.int32, (RB, D2), 1)
    masked = jnp.where(col < NCLS, logits, -1e30)
    m = jnp.max(masked, axis=1, keepdims=True)
    s = jnp.sum(jnp.exp(masked - m), axis=1, keepdims=True)
    o_ref[...] = logits - m - jnp.log(s)


def kernel(x_text_feat, edge_index, W1, b1, W2, b2):
    f32 = jnp.float32
    src = edge_index[0].astype(jnp.int32)
    dst = edge_index[1].astype(jnp.int32)
    padv = N + (jnp.arange(EP - E, dtype=jnp.int32) % NP)
    src_p = jnp.concatenate([src, padv]).reshape(G, 128)
    dst_p = jnp.concatenate([dst, padv]).reshape(G, 128)

    zN = jnp.zeros((N2,), f32)
    z1 = jnp.zeros((ROWS_T, HALF), f32)
    z2 = jnp.zeros((ROWS_T, D2), f32)
    b1t = jnp.tile(b1.astype(f32).reshape(1, D_HID), (8, 1))
    W2p = jnp.zeros((D_HID, D2), f32).at[:, :NCLS].set(W2.astype(f32))
    b2t = jnp.tile(jnp.pad(b2.astype(f32), (0, D2 - NCLS)).reshape(1, D2),
                   (8, 1))

    hist = _deg_kernel(dst_p.reshape(EP), zN).reshape(NW, N2)

    h1 = pl.pallas_call(
        _mm_body,
        grid=(GRID,),
        in_specs=[
            pl.BlockSpec((RB, D_IN), lambda i: (i, 0)),
            pl.BlockSpec((D_IN, D_HID), lambda i: (0, 0)),
        ],
        out_specs=pl.BlockSpec((RB, D_HID), lambda i: (i, 0)),
        out_shape=jax.ShapeDtypeStruct((N, D_HID), f32),
    )(x_text_feat, W1)

    p0, p1, dinv = pl.pallas_call(
        _scale_body,
        grid=(GRID,),
        in_specs=[
            pl.BlockSpec((RB, D_HID), lambda i: (i, 0)),
            pl.BlockSpec((NW, RB), lambda i: (0, i)),
        ],
        out_specs=[
            pl.BlockSpec((RB, HALF), lambda i: (i, 0)),
            pl.BlockSpec((RB, HALF), lambda i: (i, 0)),
            pl.BlockSpec((RB, 1), lambda i: (i, 0)),
        ],
        out_shape=[
            jax.ShapeDtypeStruct((N2, HALF), f32),
            jax.ShapeDtypeStruct((N2, HALF), f32),
            jax.ShapeDtypeStruct((N, 1), f32),
        ],
    )(h1, hist)

    a0, a1 = _scat1_kernel(src_p, dst_p, p0, p1, z1)

    p2 = pl.pallas_call(
        _fuse2_body,
        grid=(GRID,),
        in_specs=[
            pl.BlockSpec((RB, HALF), lambda i: (i, 0)),
            pl.BlockSpec((RB, HALF), lambda i: (i, 0)),
            pl.BlockSpec((RB, HALF), lambda i: (i, 0)),
            pl.BlockSpec((RB, HALF), lambda i: (i, 0)),
            pl.BlockSpec((RB, 1), lambda i: (i, 0)),
            pl.BlockSpec((8, D_HID), lambda i: (0, 0)),
            pl.BlockSpec((D_HID, D2), lambda i: (0, 0)),
        ],
        out_specs=pl.BlockSpec((RB, D2), lambda i: (i, 0)),
        out_shape=jax.ShapeDtypeStruct((N2, D2), f32),
    )(a0, a1, p0, p1, dinv, b1t, W2p)

    q0, q1 = _scat2_kernel(src_p, dst_p, p2, z2)

    out = pl.pallas_call(
        _fuse3_body,
        grid=(GRID,),
        in_specs=[
            pl.BlockSpec((RB, D2), lambda i: (i, 0)),
            pl.BlockSpec((RB, D2), lambda i: (i, 0)),
            pl.BlockSpec((RB, D2), lambda i: (i, 0)),
            pl.BlockSpec((RB, 1), lambda i: (i, 0)),
            pl.BlockSpec((8, D2), lambda i: (0, 0)),
        ],
        out_specs=pl.BlockSpec((RB, D2), lambda i: (i, 0)),
        out_shape=jax.ShapeDtypeStruct((N, D2), f32),
    )(q0, q1, p2, dinv, b2t)

    return out[:, :NCLS]


# speedup vs baseline: 30.7665x; 1.4892x over previous
import functools

import jax
import jax.numpy as jnp
from jax import lax
from jax.experimental import pallas as pl
from jax.experimental.pallas import tpu as pltpu
from jax.experimental.pallas import tpu_sc as plsc

N = 50000
E = 800000
D_IN = 768
D_HID = 64
NCLS = 7

NP = 48
N2 = N + NP
EP = 819200
G = EP // 128
HALF = 32
D2 = 16

NSC = 2
NT = 16
NW = NSC * NT

ROWS_T = N2 // NT

TG1 = G // NT
SCH1 = 40
NSC1 = TG1 // SCH1

TG2 = G // NW
SCH2 = 40
NSC2 = TG2 // SCH2

EPTD = EP // NW
NGD = EPTD // 16

NBUF = 4

RB = 512
GRID = (N + RB - 1) // RB

_mesh = plsc.VectorSubcoreMesh(core_axis_name="c", subcore_axis_name="s")
_SC_PARAMS = pltpu.CompilerParams(needs_layout_passes=False,
                                  use_tc_tiling_on_sc=False)


@functools.partial(
    pl.kernel,
    out_type=jax.ShapeDtypeStruct((NW * N2,), jnp.float32),
    mesh=_mesh,
    scratch_types=[
        pltpu.VMEM((EPTD,), jnp.int32),
        pltpu.VMEM((N2,), jnp.float32),
    ],
    compiler_params=_SC_PARAMS,
)
def _deg_kernel(dst_hbm, zeros_hbm, hist_hbm, dst_v, hist_v):
    cid = lax.axis_index("c")
    sid = lax.axis_index("s")
    wid = sid * NSC + cid
    pltpu.sync_copy(zeros_hbm, hist_v)
    pltpu.sync_copy(dst_hbm.at[pl.ds(wid * EPTD, EPTD)], dst_v)
    ones16 = jnp.ones((16,), jnp.float32)

    def body(i, carry):
        idx = dst_v[pl.ds(i * 16, 16)]
        plsc.addupdate_scatter(hist_v, [idx], ones16)
        return carry

    lax.fori_loop(0, NGD, body, 0)
    pltpu.sync_copy(hist_v, hist_hbm.at[pl.ds(wid * N2, N2)])


@functools.partial(
    pl.kernel,
    out_type=(
        jax.ShapeDtypeStruct((N2, HALF), jnp.float32),
        jax.ShapeDtypeStruct((N2, HALF), jnp.float32),
    ),
    mesh=_mesh,
    scratch_types=[
        pltpu.VMEM((SCH1, 128), jnp.int32),
        pltpu.VMEM((SCH1, 128), jnp.int32),
        [pltpu.VMEM((128, HALF), jnp.float32) for _ in range(NBUF)],
        pltpu.VMEM_SHARED((N2, HALF), jnp.float32),
        [pltpu.SemaphoreType.DMA for _ in range(NBUF)],
    ],
    compiler_params=_SC_PARAMS,
)
def _scat1_kernel(src_hbm, dst_hbm, p0_hbm, p1_hbm, z_hbm,
                  o0_hbm, o1_hbm, src_v, dst_v, rows, acc_s, gsem):
    cid = lax.axis_index("c")
    sid = lax.axis_index("s")
    r0 = sid * ROWS_T
    pltpu.sync_copy(z_hbm, acc_s.at[pl.ds(r0, ROWS_T)])
    plsc.subcore_barrier()

    def gath0(j, b):
        pltpu.async_copy(p0_hbm.at[src_v.at[j]], rows[b], gsem[b])

    def gath1(j, b):
        pltpu.async_copy(p1_hbm.at[src_v.at[j]], rows[b], gsem[b])

    def superchunk(t, carry):
        g0 = sid * TG1 + t * SCH1
        pltpu.sync_copy(src_hbm.at[pl.ds(g0, SCH1)], src_v)
        pltpu.sync_copy(dst_hbm.at[pl.ds(g0, SCH1)], dst_v)

        @pl.when(cid == 0)
        def _():
            for j in range(NBUF):
                gath0(j, j)
            for j in range(SCH1):
                b = j % NBUF
                pltpu.make_async_copy(p0_hbm.at[src_v.at[j]],
                                      rows[b], gsem[b]).wait()
                pltpu.sync_copy(rows[b], acc_s.at[dst_v.at[j]], add=True)
                if j + NBUF < SCH1:
                    gath0(j + NBUF, b)

        @pl.when(cid == 1)
        def _():
            for j in range(NBUF):
                gath1(j, j)
            for j in range(SCH1):
                b = j % NBUF
                pltpu.make_async_copy(p1_hbm.at[src_v.at[j]],
                                      rows[b], gsem[b]).wait()
                pltpu.sync_copy(rows[b], acc_s.at[dst_v.at[j]], add=True)
                if j + NBUF < SCH1:
                    gath1(j + NBUF, b)

        return carry

    lax.fori_loop(0, NSC1, superchunk, 0)
    plsc.subcore_barrier()

    @pl.when(cid == 0)
    def _():
        pltpu.sync_copy(acc_s.at[pl.ds(r0, ROWS_T)], o0_hbm.at[pl.ds(r0, ROWS_T)])

    @pl.when(cid == 1)
    def _():
        pltpu.sync_copy(acc_s.at[pl.ds(r0, ROWS_T)], o1_hbm.at[pl.ds(r0, ROWS_T)])


@functools.partial(
    pl.kernel,
    out_type=(
        jax.ShapeDtypeStruct((N2, D2), jnp.float32),
        jax.ShapeDtypeStruct((N2, D2), jnp.float32),
    ),
    mesh=_mesh,
    scratch_types=[
        pltpu.VMEM((SCH2, 128), jnp.int32),
        pltpu.VMEM((SCH2, 128), jnp.int32),
        [pltpu.VMEM((128, D2), jnp.float32) for _ in range(NBUF)],
        pltpu.VMEM_SHARED((N2, D2), jnp.float32),
        [pltpu.SemaphoreType.DMA for _ in range(NBUF)],
    ],
    compiler_params=_SC_PARAMS,
)
def _scat2_kernel(src_hbm, dst_hbm, p2_hbm, z_hbm,
                  q0_hbm, q1_hbm, src_v, dst_v, rows, acc_s, gsem):
    cid = lax.axis_index("c")
    sid = lax.axis_index("s")
    wid = sid * NSC + cid
    r0 = sid * ROWS_T
    pltpu.sync_copy(z_hbm, acc_s.at[pl.ds(r0, ROWS_T)])
    plsc.subcore_barrier()

    def gath(j, b):
        pltpu.async_copy(p2_hbm.at[src_v.at[j]], rows[b], gsem[b])

    def superchunk(t, carry):
        g0 = wid * TG2 + t * SCH2
        pltpu.sync_copy(src_hbm.at[pl.ds(g0, SCH2)], src_v)
        pltpu.sync_copy(dst_hbm.at[pl.ds(g0, SCH2)], dst_v)

        for j in range(NBUF):
            gath(j, j)
        for j in range(SCH2):
            b = j % NBUF
            pltpu.make_async_copy(p2_hbm.at[src_v.at[j]],
                                  rows[b], gsem[b]).wait()
            pltpu.sync_copy(rows[b], acc_s.at[dst_v.at[j]], add=True)
            if j + NBUF < SCH2:
                gath(j + NBUF, b)
        return carry

    lax.fori_loop(0, NSC2, superchunk, 0)
    plsc.subcore_barrier()

    @pl.when(cid == 0)
    def _():
        pltpu.sync_copy(acc_s.at[pl.ds(r0, ROWS_T)], q0_hbm.at[pl.ds(r0, ROWS_T)])

    @pl.when(cid == 1)
    def _():
        pltpu.sync_copy(acc_s.at[pl.ds(r0, ROWS_T)], q1_hbm.at[pl.ds(r0, ROWS_T)])


def _mm_body(x_ref, w_ref, o_ref):
    o_ref[...] = jnp.dot(x_ref[...], w_ref[...],
                         preferred_element_type=jnp.float32)


def _scale_body(h_ref, hist_ref, p0_ref, p1_ref, dinv_ref):
    deg = 1.0 + jnp.sum(hist_ref[...], axis=0)
    dinv = lax.rsqrt(deg)[:, None]
    p = h_ref[...] * dinv
    p0_ref[...] = p[:, :HALF]
    p1_ref[...] = p[:, HALF:]
    dinv_ref[...] = dinv


def _fuse2_body(a0_ref, a1_ref, p0_ref, p1_ref, dinv_ref, b1_ref, w2_ref,
                p2_ref):
    dinv = dinv_ref[...]
    x0 = jnp.maximum((a0_ref[...] + p0_ref[...]) * dinv
                     + b1_ref[0:1, :HALF], 0.0)
    x1 = jnp.maximum((a1_ref[...] + p1_ref[...]) * dinv
                     + b1_ref[0:1, HALF:], 0.0)
    h2 = (jnp.dot(x0, w2_ref[:HALF, :], preferred_element_type=jnp.float32)
          + jnp.dot(x1, w2_ref[HALF:, :], preferred_element_type=jnp.float32))
    p2_ref[...] = h2 * dinv


def _fuse3_body(q0_ref, q1_ref, p2_ref, dinv_ref, b2_ref, o_ref):
    logits = ((q0_ref[...] + q1_ref[...] + p2_ref[...]) * dinv_ref[...]
              + b2_ref[0:1, :])
    col = lax.broadcasted_iota(jnp.int32, (RB, D2), 1)
    masked = jnp.where(col < NCLS, logits, -1e30)
    m = jnp.max(masked, axis=1, keepdims=True)
    s = jnp.sum(jnp.exp(masked - m), axis=1, keepdims=True)
    o_ref[...] = logits - m - jnp.log(s)


def kernel(x_text_feat, edge_index, W1, b1, W2, b2):
    f32 = jnp.float32
    src = edge_index[0].astype(jnp.int32)
    dst = edge_index[1].astype(jnp.int32)
    padv = N + (jnp.arange(EP - E, dtype=jnp.int32) % NP)
    src_p = jnp.concatenate([src, padv]).reshape(G, 128)
    dst_p = jnp.concatenate([dst, padv]).reshape(G, 128)

    zN = jnp.zeros((N2,), f32)
    z1 = jnp.zeros((ROWS_T, HALF), f32)
    z2 = jnp.zeros((ROWS_T, D2), f32)
    b1t = jnp.tile(b1.astype(f32).reshape(1, D_HID), (8, 1))
    W2p = jnp.zeros((D_HID, D2), f32).at[:, :NCLS].set(W2.astype(f32))
    b2t = jnp.tile(jnp.pad(b2.astype(f32), (0, D2 - NCLS)).reshape(1, D2),
                   (8, 1))

    hist = _deg_kernel(dst_p.reshape(EP), zN).reshape(NW, N2)

    h1 = pl.pallas_call(
        _mm_body,
        grid=(GRID,),
        in_specs=[
            pl.BlockSpec((RB, D_IN), lambda i: (i, 0)),
            pl.BlockSpec((D_IN, D_HID), lambda i: (0, 0)),
        ],
        out_specs=pl.BlockSpec((RB, D_HID), lambda i: (i, 0)),
        out_shape=jax.ShapeDtypeStruct((N, D_HID), f32),
    )(x_text_feat, W1)

    p0, p1, dinv = pl.pallas_call(
        _scale_body,
        grid=(GRID,),
        in_specs=[
            pl.BlockSpec((RB, D_HID), lambda i: (i, 0)),
            pl.BlockSpec((NW, RB), lambda i: (0, i)),
        ],
        out_specs=[
            pl.BlockSpec((RB, HALF), lambda i: (i, 0)),
            pl.BlockSpec((RB, HALF), lambda i: (i, 0)),
            pl.BlockSpec((RB, 1), lambda i: (i, 0)),
        ],
        out_shape=[
            jax.ShapeDtypeStruct((N2, HALF), f32),
            jax.ShapeDtypeStruct((N2, HALF), f32),
            jax.ShapeDtypeStruct((N, 1), f32),
        ],
    )(h1, hist)

    a0, a1 = _scat1_kernel(src_p, dst_p, p0, p1, z1)

    p2 = pl.pallas_call(
        _fuse2_body,
        grid=(GRID,),
        in_specs=[
            pl.BlockSpec((RB, HALF), lambda i: (i, 0)),
            pl.BlockSpec((RB, HALF), lambda i: (i, 0)),
            pl.BlockSpec((RB, HALF), lambda i: (i, 0)),
            pl.BlockSpec((RB, HALF), lambda i: (i, 0)),
            pl.BlockSpec((RB, 1), lambda i: (i, 0)),
            pl.BlockSpec((8, D_HID), lambda i: (0, 0)),
            pl.BlockSpec((D_HID, D2), lambda i: (0, 0)),
        ],
        out_specs=pl.BlockSpec((RB, D2), lambda i: (i, 0)),
        out_shape=jax.ShapeDtypeStruct((N2, D2), f32),
    )(a0, a1, p0, p1, dinv, b1t, W2p)

    q0, q1 = _scat2_kernel(src_p, dst_p, p2, z2)

    out = pl.pallas_call(
        _fuse3_body,
        grid=(GRID,),
        in_specs=[
            pl.BlockSpec((RB, D2), lambda i: (i, 0)),
            pl.BlockSpec((RB, D2), lambda i: (i, 0)),
            pl.BlockSpec((RB, D2), lambda i: (i, 0)),
            pl.BlockSpec((RB, 1), lambda i: (i, 0)),
            pl.BlockSpec((8, D2), lambda i: (0, 0)),
        ],
        out_specs=pl.BlockSpec((RB, D2), lambda i: (i, 0)),
        out_shape=jax.ShapeDtypeStruct((N, D2), f32),
    )(q0, q1, p2, dinv, b2t)

    return out[:, :NCLS]


# speedup vs baseline: 37.6164x; 1.2226x over previous
import functools

import jax
import jax.numpy as jnp
from jax import lax
from jax.experimental import pallas as pl
from jax.experimental.pallas import tpu as pltpu
from jax.experimental.pallas import tpu_sc as plsc

N = 50000
E = 800000
D_IN = 768
D_HID = 64
NCLS = 7

NP = 48
N2 = N + NP
EP = 819200
G = EP // 128
HALF = 32
D2 = 16

NSC = 2
NT = 16
NW = NSC * NT

ROWS_T = N2 // NT

TG1 = G // NT
SCH1 = 40
NSC1 = TG1 // SCH1

TG2 = G // NW
SCH2 = 40
NSC2 = TG2 // SCH2

EPTD = EP // NW
NGD = EPTD // 16

NBUF1 = 4
NBUF2 = 8

RB = 1024
GRID = (N + RB - 1) // RB

_mesh = plsc.VectorSubcoreMesh(core_axis_name="c", subcore_axis_name="s")
_SC_PARAMS = pltpu.CompilerParams(needs_layout_passes=False,
                                  use_tc_tiling_on_sc=False)


@functools.partial(
    pl.kernel,
    out_type=jax.ShapeDtypeStruct((NW * N2,), jnp.float32),
    mesh=_mesh,
    scratch_types=[
        pltpu.VMEM((EPTD,), jnp.int32),
        pltpu.VMEM((N2,), jnp.float32),
    ],
    compiler_params=_SC_PARAMS,
)
def _deg_kernel(dst_hbm, zeros_hbm, hist_hbm, dst_v, hist_v):
    cid = lax.axis_index("c")
    sid = lax.axis_index("s")
    wid = sid * NSC + cid
    pltpu.sync_copy(zeros_hbm, hist_v)
    pltpu.sync_copy(dst_hbm.at[pl.ds(wid * EPTD, EPTD)], dst_v)
    ones16 = jnp.ones((16,), jnp.float32)

    def body(i, carry):
        idx = dst_v[pl.ds(i * 16, 16)]
        plsc.addupdate_scatter(hist_v, [idx], ones16)
        return carry

    lax.fori_loop(0, NGD, body, 0)
    pltpu.sync_copy(hist_v, hist_hbm.at[pl.ds(wid * N2, N2)])


@functools.partial(
    pl.kernel,
    out_type=(
        jax.ShapeDtypeStruct((N2, HALF), jnp.float32),
        jax.ShapeDtypeStruct((N2, HALF), jnp.float32),
    ),
    mesh=_mesh,
    scratch_types=[
        pltpu.VMEM((SCH1, 128), jnp.int32),
        pltpu.VMEM((SCH1, 128), jnp.int32),
        [pltpu.VMEM((128, HALF), jnp.float32) for _ in range(NBUF1)],
        pltpu.VMEM_SHARED((N2, HALF), jnp.float32),
        [pltpu.SemaphoreType.DMA for _ in range(NBUF1)],
    ],
    compiler_params=_SC_PARAMS,
)
def _scat1_kernel(src_hbm, dst_hbm, p0_hbm, p1_hbm, z_hbm,
                  o0_hbm, o1_hbm, src_v, dst_v, rows, acc_s, gsem):
    cid = lax.axis_index("c")
    sid = lax.axis_index("s")
    r0 = sid * ROWS_T
    pltpu.sync_copy(z_hbm, acc_s.at[pl.ds(r0, ROWS_T)])
    plsc.subcore_barrier()

    def gath0(j, b):
        pltpu.async_copy(p0_hbm.at[src_v.at[j]], rows[b], gsem[b])

    def gath1(j, b):
        pltpu.async_copy(p1_hbm.at[src_v.at[j]], rows[b], gsem[b])

    def superchunk(t, carry):
        g0 = sid * TG1 + t * SCH1
        pltpu.sync_copy(src_hbm.at[pl.ds(g0, SCH1)], src_v)
        pltpu.sync_copy(dst_hbm.at[pl.ds(g0, SCH1)], dst_v)

        @pl.when(cid == 0)
        def _():
            for j in range(NBUF1):
                gath0(j, j)
            for j in range(SCH1):
                b = j % NBUF1
                pltpu.make_async_copy(p0_hbm.at[src_v.at[j]],
                                      rows[b], gsem[b]).wait()
                pltpu.sync_copy(rows[b], acc_s.at[dst_v.at[j]], add=True)
                if j + NBUF1 < SCH1:
                    gath0(j + NBUF1, b)

        @pl.when(cid == 1)
        def _():
            for j in range(NBUF1):
                gath1(j, j)
            for j in range(SCH1):
                b = j % NBUF1
                pltpu.make_async_copy(p1_hbm.at[src_v.at[j]],
                                      rows[b], gsem[b]).wait()
                pltpu.sync_copy(rows[b], acc_s.at[dst_v.at[j]], add=True)
                if j + NBUF1 < SCH1:
                    gath1(j + NBUF1, b)

        return carry

    lax.fori_loop(0, NSC1, superchunk, 0)
    plsc.subcore_barrier()

    @pl.when(cid == 0)
    def _():
        pltpu.sync_copy(acc_s.at[pl.ds(r0, ROWS_T)], o0_hbm.at[pl.ds(r0, ROWS_T)])

    @pl.when(cid == 1)
    def _():
        pltpu.sync_copy(acc_s.at[pl.ds(r0, ROWS_T)], o1_hbm.at[pl.ds(r0, ROWS_T)])


@functools.partial(
    pl.kernel,
    out_type=(
        jax.ShapeDtypeStruct((N2, D2), jnp.float32),
        jax.ShapeDtypeStruct((N2, D2), jnp.float32),
    ),
    mesh=_mesh,
    scratch_types=[
        pltpu.VMEM((SCH2, 128), jnp.int32),
        pltpu.VMEM((SCH2, 128), jnp.int32),
        [pltpu.VMEM((128, D2), jnp.float32) for _ in range(NBUF2)],
        pltpu.VMEM_SHARED((N2, D2), jnp.float32),
        [pltpu.SemaphoreType.DMA for _ in range(NBUF2)],
    ],
    compiler_params=_SC_PARAMS,
)
def _scat2_kernel(src_hbm, dst_hbm, p2_hbm, z_hbm,
                  q0_hbm, q1_hbm, src_v, dst_v, rows, acc_s, gsem):
    cid = lax.axis_index("c")
    sid = lax.axis_index("s")
    wid = sid * NSC + cid
    r0 = sid * ROWS_T
    pltpu.sync_copy(z_hbm, acc_s.at[pl.ds(r0, ROWS_T)])
    plsc.subcore_barrier()

    def gath(j, b):
        pltpu.async_copy(p2_hbm.at[src_v.at[j]], rows[b], gsem[b])

    def superchunk(t, carry):
        g0 = wid * TG2 + t * SCH2
        pltpu.sync_copy(src_hbm.at[pl.ds(g0, SCH2)], src_v)
        pltpu.sync_copy(dst_hbm.at[pl.ds(g0, SCH2)], dst_v)

        for j in range(NBUF2):
            gath(j, j)
        for j in range(SCH2):
            b = j % NBUF2
            pltpu.make_async_copy(p2_hbm.at[src_v.at[j]],
                                  rows[b], gsem[b]).wait()
            pltpu.sync_copy(rows[b], acc_s.at[dst_v.at[j]], add=True)
            if j + NBUF2 < SCH2:
                gath(j + NBUF2, b)
        return carry

    lax.fori_loop(0, NSC2, superchunk, 0)
    plsc.subcore_barrier()

    @pl.when(cid == 0)
    def _():
        pltpu.sync_copy(acc_s.at[pl.ds(r0, ROWS_T)], q0_hbm.at[pl.ds(r0, ROWS_T)])

    @pl.when(cid == 1)
    def _():
        pltpu.sync_copy(acc_s.at[pl.ds(r0, ROWS_T)], q1_hbm.at[pl.ds(r0, ROWS_T)])


def _mmscale_body(x_ref, w_ref, hist_ref, p0_ref, p1_ref, dinv_ref):
    h = jnp.dot(x_ref[...], w_ref[...], preferred_element_type=jnp.float32)
    deg = 1.0 + jnp.sum(hist_ref[...], axis=0)
    dinv = lax.rsqrt(deg)[:, None]
    p = h * dinv
    p0_ref[...] = p[:, :HALF]
    p1_ref[...] = p[:, HALF:]
    dinv_ref[...] = dinv


def _fuse2_body(a0_ref, a1_ref, p0_ref, p1_ref, dinv_ref, b1_ref, w2_ref,
                p2_ref):
    dinv = dinv_ref[...]
    x0 = jnp.maximum((a0_ref[...] + p0_ref[...]) * dinv
                     + b1_ref[0:1, :HALF], 0.0)
    x1 = jnp.maximum((a1_ref[...] + p1_ref[...]) * dinv
                     + b1_ref[0:1, HALF:], 0.0)
    h2 = (jnp.dot(x0, w2_ref[:HALF, :], preferred_element_type=jnp.float32)
          + jnp.dot(x1, w2_ref[HALF:, :], preferred_element_type=jnp.float32))
    p2_ref[...] = h2 * dinv


def _fuse3_body(q0_ref, q1_ref, p2_ref, dinv_ref, b2_ref, o_ref):
    logits = ((q0_ref[...] + q1_ref[...] + p2_ref[...]) * dinv_ref[...]
              + b2_ref[0:1, :])
    col = lax.broadcasted_iota(jnp.int32, (RB, D2), 1)
    masked = jnp.where(col < NCLS, logits, -1e30)
    m = jnp.max(masked, axis=1, keepdims=True)
    s = jnp.sum(jnp.exp(masked - m), axis=1, keepdims=True)
    o_ref[...] = logits - m - jnp.log(s)


def kernel(x_text_feat, edge_index, W1, b1, W2, b2):
    f32 = jnp.float32
    src = edge_index[0].astype(jnp.int32)
    dst = edge_index[1].astype(jnp.int32)
    padv = N + (jnp.arange(EP - E, dtype=jnp.int32) % NP)
    src_p = jnp.concatenate([src, padv]).reshape(G, 128)
    dst_p = jnp.concatenate([dst, padv]).reshape(G, 128)

    zN = jnp.zeros((N2,), f32)
    z1 = jnp.zeros((ROWS_T, HALF), f32)
    z2 = jnp.zeros((ROWS_T, D2), f32)
    b1t = jnp.tile(b1.astype(f32).reshape(1, D_HID), (8, 1))
    W2p = jnp.zeros((D_HID, D2), f32).at[:, :NCLS].set(W2.astype(f32))
    b2t = jnp.tile(jnp.pad(b2.astype(f32), (0, D2 - NCLS)).reshape(1, D2),
                   (8, 1))

    hist = _deg_kernel(dst_p.reshape(EP), zN).reshape(NW, N2)

    p0, p1, dinv = pl.pallas_call(
        _mmscale_body,
        grid=(GRID,),
        in_specs=[
            pl.BlockSpec((RB, D_IN), lambda i: (i, 0)),
            pl.BlockSpec((D_IN, D_HID), lambda i: (0, 0)),
            pl.BlockSpec((NW, RB), lambda i: (0, i)),
        ],
        out_specs=[
            pl.BlockSpec((RB, HALF), lambda i: (i, 0)),
            pl.BlockSpec((RB, HALF), lambda i: (i, 0)),
            pl.BlockSpec((RB, 1), lambda i: (i, 0)),
        ],
        out_shape=[
            jax.ShapeDtypeStruct((N2, HALF), f32),
            jax.ShapeDtypeStruct((N2, HALF), f32),
            jax.ShapeDtypeStruct((N, 1), f32),
        ],
    )(x_text_feat, W1, hist)

    a0, a1 = _scat1_kernel(src_p, dst_p, p0, p1, z1)

    p2 = pl.pallas_call(
        _fuse2_body,
        grid=(GRID,),
        in_specs=[
            pl.BlockSpec((RB, HALF), lambda i: (i, 0)),
            pl.BlockSpec((RB, HALF), lambda i: (i, 0)),
            pl.BlockSpec((RB, HALF), lambda i: (i, 0)),
            pl.BlockSpec((RB, HALF), lambda i: (i, 0)),
            pl.BlockSpec((RB, 1), lambda i: (i, 0)),
            pl.BlockSpec((8, D_HID), lambda i: (0, 0)),
            pl.BlockSpec((D_HID, D2), lambda i: (0, 0)),
        ],
        out_specs=pl.BlockSpec((RB, D2), lambda i: (i, 0)),
        out_shape=jax.ShapeDtypeStruct((N2, D2), f32),
    )(a0, a1, p0, p1, dinv, b1t, W2p)

    q0, q1 = _scat2_kernel(src_p, dst_p, p2, z2)

    out = pl.pallas_call(
        _fuse3_body,
        grid=(GRID,),
        in_specs=[
            pl.BlockSpec((RB, D2), lambda i: (i, 0)),
            pl.BlockSpec((RB, D2), lambda i: (i, 0)),
            pl.BlockSpec((RB, D2), lambda i: (i, 0)),
            pl.BlockSpec((RB, 1), lambda i: (i, 0)),
            pl.BlockSpec((8, D2), lambda i: (0, 0)),
        ],
        out_specs=pl.BlockSpec((RB, D2), lambda i: (i, 0)),
        out_shape=jax.ShapeDtypeStruct((N, D2), f32),
    )(q0, q1, p2, dinv, b2t)

    return out[:, :NCLS]


# speedup vs baseline: 38.9844x; 1.0364x over previous
import functools

import jax
import jax.numpy as jnp
from jax import lax
from jax.experimental import pallas as pl
from jax.experimental.pallas import tpu as pltpu
from jax.experimental.pallas import tpu_sc as plsc

N = 50000
E = 800000
D_IN = 768
D_HID = 64
NCLS = 7

NP = 48
N2 = N + NP
EP = 819200
G = EP // 128
HALF = 32
D2 = 16

NSC = 2
NT = 16
NW = NSC * NT

ROWS_T = N2 // NT

TG1 = G // NT
SCH1 = 40
NSC1 = TG1 // SCH1

TG2 = G // NW
SCH2 = 40
NSC2 = TG2 // SCH2

EPTD = EP // NW
NGD = EPTD // 16

NBUF1 = 4
NBUF2 = 8
GLAG = 2

RB1 = 2048
GRID1 = (N + RB1 - 1) // RB1
RBF = 4096
GRIDF = (N + RBF - 1) // RBF

_mesh = plsc.VectorSubcoreMesh(core_axis_name="c", subcore_axis_name="s")
_SC_PARAMS = pltpu.CompilerParams(needs_layout_passes=False,
                                  use_tc_tiling_on_sc=False)


@functools.partial(
    pl.kernel,
    out_type=jax.ShapeDtypeStruct((NW * N2,), jnp.float32),
    mesh=_mesh,
    scratch_types=[
        pltpu.VMEM((EPTD,), jnp.int32),
        pltpu.VMEM((N2,), jnp.float32),
    ],
    compiler_params=_SC_PARAMS,
)
def _deg_kernel(dst_hbm, zeros_hbm, hist_hbm, dst_v, hist_v):
    cid = lax.axis_index("c")
    sid = lax.axis_index("s")
    wid = sid * NSC + cid
    pltpu.sync_copy(zeros_hbm, hist_v)
    pltpu.sync_copy(dst_hbm.at[pl.ds(wid * EPTD, EPTD)], dst_v)
    ones16 = jnp.ones((16,), jnp.float32)

    def body(i, carry):
        idx = dst_v[pl.ds(i * 16, 16)]
        plsc.addupdate_scatter(hist_v, [idx], ones16)
        return carry

    lax.fori_loop(0, NGD, body, 0)
    pltpu.sync_copy(hist_v, hist_hbm.at[pl.ds(wid * N2, N2)])


@functools.partial(
    pl.kernel,
    out_type=(
        jax.ShapeDtypeStruct((N2, HALF), jnp.float32),
        jax.ShapeDtypeStruct((N2, HALF), jnp.float32),
    ),
    mesh=_mesh,
    scratch_types=[
        pltpu.VMEM((SCH1, 128), jnp.int32),
        pltpu.VMEM((SCH1, 128), jnp.int32),
        [pltpu.VMEM((128, HALF), jnp.float32) for _ in range(NBUF1)],
        pltpu.VMEM_SHARED((N2, HALF), jnp.float32),
        [pltpu.SemaphoreType.DMA for _ in range(NBUF1)],
        [pltpu.SemaphoreType.DMA for _ in range(NBUF1)],
    ],
    compiler_params=_SC_PARAMS,
)
def _scat1_kernel(src_hbm, dst_hbm, p0_hbm, p1_hbm, z_hbm,
                  o0_hbm, o1_hbm, src_v, dst_v, rows, acc_s, gsem, ssem):
    cid = lax.axis_index("c")
    sid = lax.axis_index("s")
    r0 = sid * ROWS_T
    pltpu.sync_copy(z_hbm, acc_s.at[pl.ds(r0, ROWS_T)])
    plsc.subcore_barrier()

    def pipe(p_hbm):
        def superchunk(t, carry):
            g0 = sid * TG1 + t * SCH1
            pltpu.sync_copy(src_hbm.at[pl.ds(g0, SCH1)], src_v)
            pltpu.sync_copy(dst_hbm.at[pl.ds(g0, SCH1)], dst_v)
            for j in range(SCH1 + GLAG):
                b = j % NBUF1
                if j < SCH1:
                    if j >= NBUF1:
                        pltpu.make_async_copy(
                            rows[b], acc_s.at[dst_v.at[j - NBUF1]],
                            ssem[b]).wait()
                    pltpu.async_copy(p_hbm.at[src_v.at[j]], rows[b], gsem[b])
                if j >= GLAG:
                    k = j - GLAG
                    bk = k % NBUF1
                    pltpu.make_async_copy(p_hbm.at[src_v.at[k]],
                                          rows[bk], gsem[bk]).wait()
                    pltpu.async_copy(rows[bk], acc_s.at[dst_v.at[k]],
                                     ssem[bk], add=True)
            for i in range(NBUF1):
                k = SCH1 - NBUF1 + i
                b = k % NBUF1
                pltpu.make_async_copy(rows[b], acc_s.at[dst_v.at[k]],
                                      ssem[b]).wait()
            return carry
        return superchunk

    @pl.when(cid == 0)
    def _():
        lax.fori_loop(0, NSC1, pipe(p0_hbm), 0)

    @pl.when(cid == 1)
    def _():
        lax.fori_loop(0, NSC1, pipe(p1_hbm), 0)

    plsc.subcore_barrier()

    @pl.when(cid == 0)
    def _():
        pltpu.sync_copy(acc_s.at[pl.ds(r0, ROWS_T)], o0_hbm.at[pl.ds(r0, ROWS_T)])

    @pl.when(cid == 1)
    def _():
        pltpu.sync_copy(acc_s.at[pl.ds(r0, ROWS_T)], o1_hbm.at[pl.ds(r0, ROWS_T)])


@functools.partial(
    pl.kernel,
    out_type=(
        jax.ShapeDtypeStruct((N2, D2), jnp.float32),
        jax.ShapeDtypeStruct((N2, D2), jnp.float32),
    ),
    mesh=_mesh,
    scratch_types=[
        pltpu.VMEM((SCH2, 128), jnp.int32),
        pltpu.VMEM((SCH2, 128), jnp.int32),
        [pltpu.VMEM((128, D2), jnp.float32) for _ in range(NBUF2)],
        pltpu.VMEM_SHARED((N2, D2), jnp.float32),
        [pltpu.SemaphoreType.DMA for _ in range(NBUF2)],
        [pltpu.SemaphoreType.DMA for _ in range(NBUF2)],
    ],
    compiler_params=_SC_PARAMS,
)
def _scat2_kernel(src_hbm, dst_hbm, p2_hbm, z_hbm,
                  q0_hbm, q1_hbm, src_v, dst_v, rows, acc_s, gsem, ssem):
    cid = lax.axis_index("c")
    sid = lax.axis_index("s")
    wid = sid * NSC + cid
    r0 = sid * ROWS_T
    pltpu.sync_copy(z_hbm, acc_s.at[pl.ds(r0, ROWS_T)])
    plsc.subcore_barrier()

    def superchunk(t, carry):
        g0 = wid * TG2 + t * SCH2
        pltpu.sync_copy(src_hbm.at[pl.ds(g0, SCH2)], src_v)
        pltpu.sync_copy(dst_hbm.at[pl.ds(g0, SCH2)], dst_v)
        for j in range(SCH2 + GLAG):
            b = j % NBUF2
            if j < SCH2:
                if j >= NBUF2:
                    pltpu.make_async_copy(
                        rows[b], acc_s.at[dst_v.at[j - NBUF2]],
                        ssem[b]).wait()
                pltpu.async_copy(p2_hbm.at[src_v.at[j]], rows[b], gsem[b])
            if j >= GLAG:
                k = j - GLAG
                bk = k % NBUF2
                pltpu.make_async_copy(p2_hbm.at[src_v.at[k]],
                                      rows[bk], gsem[bk]).wait()
                pltpu.async_copy(rows[bk], acc_s.at[dst_v.at[k]],
                                 ssem[bk], add=True)
        for i in range(NBUF2):
            k = SCH2 - NBUF2 + i
            b = k % NBUF2
            pltpu.make_async_copy(rows[b], acc_s.at[dst_v.at[k]],
                                  ssem[b]).wait()
        return carry

    lax.fori_loop(0, NSC2, superchunk, 0)
    plsc.subcore_barrier()

    @pl.when(cid == 0)
    def _():
        pltpu.sync_copy(acc_s.at[pl.ds(r0, ROWS_T)], q0_hbm.at[pl.ds(r0, ROWS_T)])

    @pl.when(cid == 1)
    def _():
        pltpu.sync_copy(acc_s.at[pl.ds(r0, ROWS_T)], q1_hbm.at[pl.ds(r0, ROWS_T)])


def _mmscale_body(x_ref, w_ref, hist_ref, p0_ref, p1_ref, dinv_ref):
    h = jnp.dot(x_ref[...], w_ref[...], preferred_element_type=jnp.float32)
    deg = 1.0 + jnp.sum(hist_ref[...], axis=0)
    dinv = lax.rsqrt(deg)[:, None]
    p = h * dinv
    p0_ref[...] = p[:, :HALF]
    p1_ref[...] = p[:, HALF:]
    dinv_ref[...] = dinv


def _fuse2_body(a0_ref, a1_ref, p0_ref, p1_ref, dinv_ref, b1_ref, w2_ref,
                p2_ref):
    dinv = dinv_ref[...]
    x0 = jnp.maximum((a0_ref[...] + p0_ref[...]) * dinv
                     + b1_ref[0:1, :HALF], 0.0)
    x1 = jnp.maximum((a1_ref[...] + p1_ref[...]) * dinv
                     + b1_ref[0:1, HALF:], 0.0)
    h2 = (jnp.dot(x0, w2_ref[:HALF, :], preferred_element_type=jnp.float32)
          + jnp.dot(x1, w2_ref[HALF:, :], preferred_element_type=jnp.float32))
    p2_ref[...] = h2 * dinv


def _fuse3_body(q0_ref, q1_ref, p2_ref, dinv_ref, b2_ref, o_ref):
    logits = ((q0_ref[...] + q1_ref[...] + p2_ref[...]) * dinv_ref[...]
              + b2_ref[0:1, :])
    col = lax.broadcasted_iota(jnp.int32, (RBF, D2), 1)
    masked = jnp.where(col < NCLS, logits, -1e30)
    m = jnp.max(masked, axis=1, keepdims=True)
    s = jnp.sum(jnp.exp(masked - m), axis=1, keepdims=True)
    o_ref[...] = logits - m - jnp.log(s)


def kernel(x_text_feat, edge_index, W1, b1, W2, b2):
    f32 = jnp.float32
    src = edge_index[0].astype(jnp.int32)
    dst = edge_index[1].astype(jnp.int32)
    padv = N + (jnp.arange(EP - E, dtype=jnp.int32) % NP)
    src_p = jnp.concatenate([src, padv]).reshape(G, 128)
    dst_p = jnp.concatenate([dst, padv]).reshape(G, 128)

    zN = jnp.zeros((N2,), f32)
    z1 = jnp.zeros((ROWS_T, HALF), f32)
    z2 = jnp.zeros((ROWS_T, D2), f32)
    b1t = jnp.tile(b1.astype(f32).reshape(1, D_HID), (8, 1))
    W2p = jnp.zeros((D_HID, D2), f32).at[:, :NCLS].set(W2.astype(f32))
    b2t = jnp.tile(jnp.pad(b2.astype(f32), (0, D2 - NCLS)).reshape(1, D2),
                   (8, 1))

    hist = _deg_kernel(dst_p.reshape(EP), zN).reshape(NW, N2)

    p0, p1, dinv = pl.pallas_call(
        _mmscale_body,
        grid=(GRID1,),
        in_specs=[
            pl.BlockSpec((RB1, D_IN), lambda i: (i, 0)),
            pl.BlockSpec((D_IN, D_HID), lambda i: (0, 0)),
            pl.BlockSpec((NW, RB1), lambda i: (0, i)),
        ],
        out_specs=[
            pl.BlockSpec((RB1, HALF), lambda i: (i, 0)),
            pl.BlockSpec((RB1, HALF), lambda i: (i, 0)),
            pl.BlockSpec((RB1, 1), lambda i: (i, 0)),
        ],
        out_shape=[
            jax.ShapeDtypeStruct((N2, HALF), f32),
            jax.ShapeDtypeStruct((N2, HALF), f32),
            jax.ShapeDtypeStruct((N, 1), f32),
        ],
    )(x_text_feat, W1, hist)

    a0, a1 = _scat1_kernel(src_p, dst_p, p0, p1, z1)

    p2 = pl.pallas_call(
        _fuse2_body,
        grid=(GRIDF,),
        in_specs=[
            pl.BlockSpec((RBF, HALF), lambda i: (i, 0)),
            pl.BlockSpec((RBF, HALF), lambda i: (i, 0)),
            pl.BlockSpec((RBF, HALF), lambda i: (i, 0)),
            pl.BlockSpec((RBF, HALF), lambda i: (i, 0)),
            pl.BlockSpec((RBF, 1), lambda i: (i, 0)),
            pl.BlockSpec((8, D_HID), lambda i: (0, 0)),
            pl.BlockSpec((D_HID, D2), lambda i: (0, 0)),
        ],
        out_specs=pl.BlockSpec((RBF, D2), lambda i: (i, 0)),
        out_shape=jax.ShapeDtypeStruct((N2, D2), f32),
    )(a0, a1, p0, p1, dinv, b1t, W2p)

    q0, q1 = _scat2_kernel(src_p, dst_p, p2, z2)

    out = pl.pallas_call(
        _fuse3_body,
        grid=(GRIDF,),
        in_specs=[
            pl.BlockSpec((RBF, D2), lambda i: (i, 0)),
            pl.BlockSpec((RBF, D2), lambda i: (i, 0)),
            pl.BlockSpec((RBF, D2), lambda i: (i, 0)),
            pl.BlockSpec((RBF, 1), lambda i: (i, 0)),
            pl.BlockSpec((8, D2), lambda i: (0, 0)),
        ],
        out_specs=pl.BlockSpec((RBF, D2), lambda i: (i, 0)),
        out_shape=jax.ShapeDtypeStruct((N, D2), f32),
    )(q0, q1, p2, dinv, b2t)

    return out[:, :NCLS]


# speedup vs baseline: 43.9011x; 1.1261x over previous
import functools

import jax
import jax.numpy as jnp
from jax import lax
from jax.experimental import pallas as pl
from jax.experimental.pallas import tpu as pltpu
from jax.experimental.pallas import tpu_sc as plsc

N = 50000
E = 800000
D_IN = 768
D_HID = 64
NCLS = 7

NP = 48
N2 = N + NP
EP = 819200
G = EP // 128
HALF = 32
D2 = 16

NSC = 2
NT = 16
NW = NSC * NT

ROWS_T = N2 // NT

TG1 = G // NT
SCH1 = 40
NSC1 = TG1 // SCH1

TG2 = G // NW
SCH2 = 40
NSC2 = TG2 // SCH2

EPTD = EP // NW
NGD = EPTD // 16

NBUF1 = 8
NBUF2 = 8

RB1 = 2048
GRID1 = (N + RB1 - 1) // RB1
RBF = 4096
GRIDF = (N + RBF - 1) // RBF

_mesh = plsc.VectorSubcoreMesh(core_axis_name="c", subcore_axis_name="s")
_SC_PARAMS = pltpu.CompilerParams(needs_layout_passes=False,
                                  use_tc_tiling_on_sc=False)


@functools.partial(
    pl.kernel,
    out_type=jax.ShapeDtypeStruct((NW * N2,), jnp.float32),
    mesh=_mesh,
    scratch_types=[
        pltpu.VMEM((EPTD,), jnp.int32),
        pltpu.VMEM((N2,), jnp.float32),
    ],
    compiler_params=_SC_PARAMS,
)
def _deg_kernel(dst_hbm, zeros_hbm, hist_hbm, dst_v, hist_v):
    cid = lax.axis_index("c")
    sid = lax.axis_index("s")
    wid = sid * NSC + cid
    pltpu.sync_copy(zeros_hbm, hist_v)
    pltpu.sync_copy(dst_hbm.at[pl.ds(wid * EPTD, EPTD)], dst_v)
    ones16 = jnp.ones((16,), jnp.float32)

    def body(i, carry):
        idx = dst_v[pl.ds(i * 16, 16)]
        plsc.addupdate_scatter(hist_v, [idx], ones16)
        return carry

    lax.fori_loop(0, NGD, body, 0)
    pltpu.sync_copy(hist_v, hist_hbm.at[pl.ds(wid * N2, N2)])


@functools.partial(
    pl.kernel,
    out_type=(
        jax.ShapeDtypeStruct((N2, HALF), jnp.bfloat16),
        jax.ShapeDtypeStruct((N2, HALF), jnp.bfloat16),
    ),
    mesh=_mesh,
    scratch_types=[
        pltpu.VMEM((SCH1, 128), jnp.int32),
        pltpu.VMEM((SCH1, 128), jnp.int32),
        [pltpu.VMEM((128, HALF), jnp.bfloat16) for _ in range(NBUF1)],
        pltpu.VMEM_SHARED((N2, HALF), jnp.bfloat16),
        [pltpu.SemaphoreType.DMA for _ in range(NBUF1)],
    ],
    compiler_params=_SC_PARAMS,
)
def _scat1_kernel(src_hbm, dst_hbm, p0_hbm, p1_hbm, z_hbm,
                  o0_hbm, o1_hbm, src_v, dst_v, rows, acc_s, gsem):
    cid = lax.axis_index("c")
    sid = lax.axis_index("s")
    r0 = sid * ROWS_T
    pltpu.sync_copy(z_hbm, acc_s.at[pl.ds(r0, ROWS_T)])
    plsc.subcore_barrier()

    def pipe(p_hbm):
        def superchunk(t, carry):
            g0 = sid * TG1 + t * SCH1
            pltpu.sync_copy(src_hbm.at[pl.ds(g0, SCH1)], src_v)
            pltpu.sync_copy(dst_hbm.at[pl.ds(g0, SCH1)], dst_v)
            for j in range(NBUF1):
                pltpu.async_copy(p_hbm.at[src_v.at[j]], rows[j], gsem[j])
            for j in range(SCH1):
                b = j % NBUF1
                pltpu.make_async_copy(p_hbm.at[src_v.at[j]],
                                      rows[b], gsem[b]).wait()
                pltpu.sync_copy(rows[b], acc_s.at[dst_v.at[j]], add=True)
                if j + NBUF1 < SCH1:
                    pltpu.async_copy(p_hbm.at[src_v.at[j + NBUF1]],
                                     rows[b], gsem[b])
            return carry
        return superchunk

    @pl.when(cid == 0)
    def _():
        lax.fori_loop(0, NSC1, pipe(p0_hbm), 0)

    @pl.when(cid == 1)
    def _():
        lax.fori_loop(0, NSC1, pipe(p1_hbm), 0)

    plsc.subcore_barrier()

    @pl.when(cid == 0)
    def _():
        pltpu.sync_copy(acc_s.at[pl.ds(r0, ROWS_T)], o0_hbm.at[pl.ds(r0, ROWS_T)])

    @pl.when(cid == 1)
    def _():
        pltpu.sync_copy(acc_s.at[pl.ds(r0, ROWS_T)], o1_hbm.at[pl.ds(r0, ROWS_T)])


@functools.partial(
    pl.kernel,
    out_type=(
        jax.ShapeDtypeStruct((N2, D2), jnp.float32),
        jax.ShapeDtypeStruct((N2, D2), jnp.float32),
    ),
    mesh=_mesh,
    scratch_types=[
        pltpu.VMEM((SCH2, 128), jnp.int32),
        pltpu.VMEM((SCH2, 128), jnp.int32),
        [pltpu.VMEM((128, D2), jnp.float32) for _ in range(NBUF2)],
        pltpu.VMEM_SHARED((N2, D2), jnp.float32),
        [pltpu.SemaphoreType.DMA for _ in range(NBUF2)],
    ],
    compiler_params=_SC_PARAMS,
)
def _scat2_kernel(src_hbm, dst_hbm, p2_hbm, z_hbm,
                  q0_hbm, q1_hbm, src_v, dst_v, rows, acc_s, gsem):
    cid = lax.axis_index("c")
    sid = lax.axis_index("s")
    wid = sid * NSC + cid
    r0 = sid * ROWS_T
    pltpu.sync_copy(z_hbm, acc_s.at[pl.ds(r0, ROWS_T)])
    plsc.subcore_barrier()

    def superchunk(t, carry):
        g0 = wid * TG2 + t * SCH2
        pltpu.sync_copy(src_hbm.at[pl.ds(g0, SCH2)], src_v)
        pltpu.sync_copy(dst_hbm.at[pl.ds(g0, SCH2)], dst_v)
        for j in range(NBUF2):
            pltpu.async_copy(p2_hbm.at[src_v.at[j]], rows[j], gsem[j])
        for j in range(SCH2):
            b = j % NBUF2
            pltpu.make_async_copy(p2_hbm.at[src_v.at[j]],
                                  rows[b], gsem[b]).wait()
            pltpu.sync_copy(rows[b], acc_s.at[dst_v.at[j]], add=True)
            if j + NBUF2 < SCH2:
                pltpu.async_copy(p2_hbm.at[src_v.at[j + NBUF2]],
                                 rows[b], gsem[b])
        return carry

    lax.fori_loop(0, NSC2, superchunk, 0)
    plsc.subcore_barrier()

    @pl.when(cid == 0)
    def _():
        pltpu.sync_copy(acc_s.at[pl.ds(r0, ROWS_T)], q0_hbm.at[pl.ds(r0, ROWS_T)])

    @pl.when(cid == 1)
    def _():
        pltpu.sync_copy(acc_s.at[pl.ds(r0, ROWS_T)], q1_hbm.at[pl.ds(r0, ROWS_T)])


def _mmscale_body(x_ref, w_ref, hist_ref, p0_ref, p1_ref, dinv_ref):
    h = jnp.dot(x_ref[...], w_ref[...], preferred_element_type=jnp.float32)
    deg = 1.0 + jnp.sum(hist_ref[...], axis=0)
    dinv = lax.rsqrt(deg)[:, None]
    p = h * dinv
    p0_ref[...] = p[:, :HALF].astype(jnp.bfloat16)
    p1_ref[...] = p[:, HALF:].astype(jnp.bfloat16)
    dinv_ref[...] = dinv


def _fuse2_body(a0_ref, a1_ref, p0_ref, p1_ref, dinv_ref, b1_ref, w2_ref,
                p2_ref):
    f32 = jnp.float32
    dinv = dinv_ref[...]
    x0 = jnp.maximum((a0_ref[...].astype(f32) + p0_ref[...].astype(f32))
                     * dinv + b1_ref[0:1, :HALF], 0.0)
    x1 = jnp.maximum((a1_ref[...].astype(f32) + p1_ref[...].astype(f32))
                     * dinv + b1_ref[0:1, HALF:], 0.0)
    h2 = (jnp.dot(x0, w2_ref[:HALF, :], preferred_element_type=f32)
          + jnp.dot(x1, w2_ref[HALF:, :], preferred_element_type=f32))
    p2_ref[...] = h2 * dinv


def _fuse3_body(q0_ref, q1_ref, p2_ref, dinv_ref, b2_ref, o_ref):
    logits = ((q0_ref[...] + q1_ref[...] + p2_ref[...]) * dinv_ref[...]
              + b2_ref[0:1, :])
    col = lax.broadcasted_iota(jnp.int32, (RBF, D2), 1)
    masked = jnp.where(col < NCLS, logits, -1e30)
    m = jnp.max(masked, axis=1, keepdims=True)
    s = jnp.sum(jnp.exp(masked - m), axis=1, keepdims=True)
    o_ref[...] = logits - m - jnp.log(s)


def kernel(x_text_feat, edge_index, W1, b1, W2, b2):
    f32 = jnp.float32
    src = edge_index[0].astype(jnp.int32)
    dst = edge_index[1].astype(jnp.int32)
    padv = N + (jnp.arange(EP - E, dtype=jnp.int32) % NP)
    src_p = jnp.concatenate([src, padv]).reshape(G, 128)
    dst_p = jnp.concatenate([dst, padv]).reshape(G, 128)

    zN = jnp.zeros((N2,), f32)
    z1 = jnp.zeros((ROWS_T, HALF), jnp.bfloat16)
    z2 = jnp.zeros((ROWS_T, D2), f32)
    b1t = jnp.tile(b1.astype(f32).reshape(1, D_HID), (8, 1))
    W2p = jnp.zeros((D_HID, D2), f32).at[:, :NCLS].set(W2.astype(f32))
    b2t = jnp.tile(jnp.pad(b2.astype(f32), (0, D2 - NCLS)).reshape(1, D2),
                   (8, 1))

    hist = _deg_kernel(dst_p.reshape(EP), zN).reshape(NW, N2)

    p0, p1, dinv = pl.pallas_call(
        _mmscale_body,
        grid=(GRID1,),
        in_specs=[
            pl.BlockSpec((RB1, D_IN), lambda i: (i, 0)),
            pl.BlockSpec((D_IN, D_HID), lambda i: (0, 0)),
            pl.BlockSpec((NW, RB1), lambda i: (0, i)),
        ],
        out_specs=[
            pl.BlockSpec((RB1, HALF), lambda i: (i, 0)),
            pl.BlockSpec((RB1, HALF), lambda i: (i, 0)),
            pl.BlockSpec((RB1, 1), lambda i: (i, 0)),
        ],
        out_shape=[
            jax.ShapeDtypeStruct((N2, HALF), jnp.bfloat16),
            jax.ShapeDtypeStruct((N2, HALF), jnp.bfloat16),
            jax.ShapeDtypeStruct((N, 1), f32),
        ],
    )(x_text_feat, W1, hist)

    a0, a1 = _scat1_kernel(src_p, dst_p, p0, p1, z1)

    p2 = pl.pallas_call(
        _fuse2_body,
        grid=(GRIDF,),
        in_specs=[
            pl.BlockSpec((RBF, HALF), lambda i: (i, 0)),
            pl.BlockSpec((RBF, HALF), lambda i: (i, 0)),
            pl.BlockSpec((RBF, HALF), lambda i: (i, 0)),
            pl.BlockSpec((RBF, HALF), lambda i: (i, 0)),
            pl.BlockSpec((RBF, 1), lambda i: (i, 0)),
            pl.BlockSpec((8, D_HID), lambda i: (0, 0)),
            pl.BlockSpec((D_HID, D2), lambda i: (0, 0)),
        ],
        out_specs=pl.BlockSpec((RBF, D2), lambda i: (i, 0)),
        out_shape=jax.ShapeDtypeStruct((N2, D2), f32),
    )(a0, a1, p0, p1, dinv, b1t, W2p)

    q0, q1 = _scat2_kernel(src_p, dst_p, p2, z2)

    out = pl.pallas_call(
        _fuse3_body,
        grid=(GRIDF,),
        in_specs=[
            pl.BlockSpec((RBF, D2), lambda i: (i, 0)),
            pl.BlockSpec((RBF, D2), lambda i: (i, 0)),
            pl.BlockSpec((RBF, D2), lambda i: (i, 0)),
            pl.BlockSpec((RBF, 1), lambda i: (i, 0)),
            pl.BlockSpec((8, D2), lambda i: (0, 0)),
        ],
        out_specs=pl.BlockSpec((RBF, D2), lambda i: (i, 0)),
        out_shape=jax.ShapeDtypeStruct((N, D2), f32),
    )(q0, q1, p2, dinv, b2t)

    return out[:, :NCLS]


# speedup vs baseline: 44.3049x; 1.0092x over previous
import functools

import jax
import jax.numpy as jnp
from jax import lax
from jax.experimental import pallas as pl
from jax.experimental.pallas import tpu as pltpu
from jax.experimental.pallas import tpu_sc as plsc

N = 50000
E = 800000
D_IN = 768
D_HID = 64
NCLS = 7

NP = 48
N2 = N + NP
EP = 819200
G = EP // 128
HALF = 32
D2 = 16

NSC = 2
NT = 16
NW = NSC * NT

ROWS_T = N2 // NT

TG1 = G // NT
SCH1 = 80
NSC1 = TG1 // SCH1

TG2 = G // NW
SCH2 = 40
NSC2 = TG2 // SCH2

EPTD = EP // NW
NGD = EPTD // 16

NBUF1 = 8
NBUF2 = 8

RB1 = 2048
GRID1 = (N + RB1 - 1) // RB1
RBF = 4096
GRIDF = (N + RBF - 1) // RBF

_mesh = plsc.VectorSubcoreMesh(core_axis_name="c", subcore_axis_name="s")
_SC_PARAMS = pltpu.CompilerParams(needs_layout_passes=False,
                                  use_tc_tiling_on_sc=False)


@functools.partial(
    pl.kernel,
    out_type=jax.ShapeDtypeStruct((NW * N2,), jnp.float32),
    mesh=_mesh,
    scratch_types=[
        pltpu.VMEM((EPTD,), jnp.int32),
        pltpu.VMEM((N2,), jnp.float32),
    ],
    compiler_params=_SC_PARAMS,
)
def _deg_kernel(dst_hbm, zeros_hbm, hist_hbm, dst_v, hist_v):
    cid = lax.axis_index("c")
    sid = lax.axis_index("s")
    wid = sid * NSC + cid
    pltpu.sync_copy(zeros_hbm, hist_v)
    pltpu.sync_copy(dst_hbm.at[pl.ds(wid * EPTD, EPTD)], dst_v)
    ones16 = jnp.ones((16,), jnp.float32)

    def body(i, carry):
        for u in range(4):
            idx = dst_v[pl.ds(i * 64 + u * 16, 16)]
            plsc.addupdate_scatter(hist_v, [idx], ones16)
        return carry

    lax.fori_loop(0, NGD // 4, body, 0)
    pltpu.sync_copy(hist_v, hist_hbm.at[pl.ds(wid * N2, N2)])


@functools.partial(
    pl.kernel,
    out_type=(
        jax.ShapeDtypeStruct((N2, HALF), jnp.bfloat16),
        jax.ShapeDtypeStruct((N2, HALF), jnp.bfloat16),
    ),
    mesh=_mesh,
    scratch_types=[
        pltpu.VMEM((SCH1, 128), jnp.int32),
        pltpu.VMEM((SCH1, 128), jnp.int32),
        [pltpu.VMEM((128, HALF), jnp.bfloat16) for _ in range(NBUF1)],
        pltpu.VMEM_SHARED((N2, HALF), jnp.bfloat16),
        [pltpu.SemaphoreType.DMA for _ in range(NBUF1)],
    ],
    compiler_params=_SC_PARAMS,
)
def _scat1_kernel(src_hbm, dst_hbm, p0_hbm, p1_hbm, z_hbm,
                  o0_hbm, o1_hbm, src_v, dst_v, rows, acc_s, gsem):
    cid = lax.axis_index("c")
    sid = lax.axis_index("s")
    r0 = sid * ROWS_T
    pltpu.sync_copy(z_hbm, acc_s.at[pl.ds(r0, ROWS_T)])
    plsc.subcore_barrier()

    def pipe(p_hbm):
        def superchunk(t, carry):
            g0 = sid * TG1 + t * SCH1
            pltpu.sync_copy(src_hbm.at[pl.ds(g0, SCH1)], src_v)
            pltpu.sync_copy(dst_hbm.at[pl.ds(g0, SCH1)], dst_v)
            for j in range(NBUF1):
                pltpu.async_copy(p_hbm.at[src_v.at[j]], rows[j], gsem[j])
            for j in range(SCH1):
                b = j % NBUF1
                pltpu.make_async_copy(p_hbm.at[src_v.at[j]],
                                      rows[b], gsem[b]).wait()
                pltpu.sync_copy(rows[b], acc_s.at[dst_v.at[j]], add=True)
                if j + NBUF1 < SCH1:
                    pltpu.async_copy(p_hbm.at[src_v.at[j + NBUF1]],
                                     rows[b], gsem[b])
            return carry
        return superchunk

    @pl.when(cid == 0)
    def _():
        lax.fori_loop(0, NSC1, pipe(p0_hbm), 0)

    @pl.when(cid == 1)
    def _():
        lax.fori_loop(0, NSC1, pipe(p1_hbm), 0)

    plsc.subcore_barrier()

    @pl.when(cid == 0)
    def _():
        pltpu.sync_copy(acc_s.at[pl.ds(r0, ROWS_T)], o0_hbm.at[pl.ds(r0, ROWS_T)])

    @pl.when(cid == 1)
    def _():
        pltpu.sync_copy(acc_s.at[pl.ds(r0, ROWS_T)], o1_hbm.at[pl.ds(r0, ROWS_T)])


@functools.partial(
    pl.kernel,
    out_type=(
        jax.ShapeDtypeStruct((N2, D2), jnp.float32),
        jax.ShapeDtypeStruct((N2, D2), jnp.float32),
    ),
    mesh=_mesh,
    scratch_types=[
        pltpu.VMEM((SCH2, 128), jnp.int32),
        pltpu.VMEM((SCH2, 128), jnp.int32),
        [pltpu.VMEM((128, D2), jnp.float32) for _ in range(NBUF2)],
        pltpu.VMEM_SHARED((N2, D2), jnp.float32),
        [pltpu.SemaphoreType.DMA for _ in range(NBUF2)],
    ],
    compiler_params=_SC_PARAMS,
)
def _scat2_kernel(src_hbm, dst_hbm, p2_hbm, z_hbm,
                  q0_hbm, q1_hbm, src_v, dst_v, rows, acc_s, gsem):
    cid = lax.axis_index("c")
    sid = lax.axis_index("s")
    wid = sid * NSC + cid
    r0 = sid * ROWS_T
    pltpu.sync_copy(z_hbm, acc_s.at[pl.ds(r0, ROWS_T)])
    plsc.subcore_barrier()

    def superchunk(t, carry):
        g0 = wid * TG2 + t * SCH2
        pltpu.sync_copy(src_hbm.at[pl.ds(g0, SCH2)], src_v)
        pltpu.sync_copy(dst_hbm.at[pl.ds(g0, SCH2)], dst_v)
        for j in range(NBUF2):
            pltpu.async_copy(p2_hbm.at[src_v.at[j]], rows[j], gsem[j])
        for j in range(SCH2):
            b = j % NBUF2
            pltpu.make_async_copy(p2_hbm.at[src_v.at[j]],
                                  rows[b], gsem[b]).wait()
            pltpu.sync_copy(rows[b], acc_s.at[dst_v.at[j]], add=True)
            if j + NBUF2 < SCH2:
                pltpu.async_copy(p2_hbm.at[src_v.at[j + NBUF2]],
                                 rows[b], gsem[b])
        return carry

    lax.fori_loop(0, NSC2, superchunk, 0)
    plsc.subcore_barrier()

    @pl.when(cid == 0)
    def _():
        pltpu.sync_copy(acc_s.at[pl.ds(r0, ROWS_T)], q0_hbm.at[pl.ds(r0, ROWS_T)])

    @pl.when(cid == 1)
    def _():
        pltpu.sync_copy(acc_s.at[pl.ds(r0, ROWS_T)], q1_hbm.at[pl.ds(r0, ROWS_T)])


def _mmscale_body(x_ref, w_ref, hist_ref, p0_ref, p1_ref, dinv_ref):
    h = jnp.dot(x_ref[...], w_ref[...], preferred_element_type=jnp.float32)
    deg = 1.0 + jnp.sum(hist_ref[...], axis=0)
    dinv = lax.rsqrt(deg)[:, None]
    p = h * dinv
    p0_ref[...] = p[:, :HALF].astype(jnp.bfloat16)
    p1_ref[...] = p[:, HALF:].astype(jnp.bfloat16)
    dinv_ref[...] = dinv


def _fuse2_body(a0_ref, a1_ref, p0_ref, p1_ref, dinv_ref, b1_ref, w2_ref,
                p2_ref):
    f32 = jnp.float32
    dinv = dinv_ref[...]
    x0 = jnp.maximum((a0_ref[...].astype(f32) + p0_ref[...].astype(f32))
                     * dinv + b1_ref[0:1, :HALF], 0.0)
    x1 = jnp.maximum((a1_ref[...].astype(f32) + p1_ref[...].astype(f32))
                     * dinv + b1_ref[0:1, HALF:], 0.0)
    h2 = (jnp.dot(x0, w2_ref[:HALF, :], preferred_element_type=f32)
          + jnp.dot(x1, w2_ref[HALF:, :], preferred_element_type=f32))
    p2_ref[...] = h2 * dinv


def _fuse3_body(q0_ref, q1_ref, p2_ref, dinv_ref, b2_ref, o_ref):
    logits = ((q0_ref[...] + q1_ref[...] + p2_ref[...]) * dinv_ref[...]
              + b2_ref[0:1, :])
    col = lax.broadcasted_iota(jnp.int32, (RBF, D2), 1)
    masked = jnp.where(col < NCLS, logits, -1e30)
    m = jnp.max(masked, axis=1, keepdims=True)
    s = jnp.sum(jnp.exp(masked - m), axis=1, keepdims=True)
    o_ref[...] = (logits - m - jnp.log(s))[:, :NCLS]


def kernel(x_text_feat, edge_index, W1, b1, W2, b2):
    f32 = jnp.float32
    src = edge_index[0].astype(jnp.int32)
    dst = edge_index[1].astype(jnp.int32)
    padv = N + (jnp.arange(EP - E, dtype=jnp.int32) % NP)
    src_p = jnp.concatenate([src, padv]).reshape(G, 128)
    dst_p = jnp.concatenate([dst, padv]).reshape(G, 128)

    zN = jnp.zeros((N2,), f32)
    z1 = jnp.zeros((ROWS_T, HALF), jnp.bfloat16)
    z2 = jnp.zeros((ROWS_T, D2), f32)
    b1t = jnp.tile(b1.astype(f32).reshape(1, D_HID), (8, 1))
    W2p = jnp.zeros((D_HID, D2), f32).at[:, :NCLS].set(W2.astype(f32))
    b2t = jnp.tile(jnp.pad(b2.astype(f32), (0, D2 - NCLS)).reshape(1, D2),
                   (8, 1))

    hist = _deg_kernel(dst_p.reshape(EP), zN).reshape(NW, N2)

    p0, p1, dinv = pl.pallas_call(
        _mmscale_body,
        grid=(GRID1,),
        in_specs=[
            pl.BlockSpec((RB1, D_IN), lambda i: (i, 0)),
            pl.BlockSpec((D_IN, D_HID), lambda i: (0, 0)),
            pl.BlockSpec((NW, RB1), lambda i: (0, i)),
        ],
        out_specs=[
            pl.BlockSpec((RB1, HALF), lambda i: (i, 0)),
            pl.BlockSpec((RB1, HALF), lambda i: (i, 0)),
            pl.BlockSpec((RB1, 1), lambda i: (i, 0)),
        ],
        out_shape=[
            jax.ShapeDtypeStruct((N2, HALF), jnp.bfloat16),
            jax.ShapeDtypeStruct((N2, HALF), jnp.bfloat16),
            jax.ShapeDtypeStruct((N, 1), f32),
        ],
    )(x_text_feat, W1, hist)

    a0, a1 = _scat1_kernel(src_p, dst_p, p0, p1, z1)

    p2 = pl.pallas_call(
        _fuse2_body,
        grid=(GRIDF,),
        in_specs=[
            pl.BlockSpec((RBF, HALF), lambda i: (i, 0)),
            pl.BlockSpec((RBF, HALF), lambda i: (i, 0)),
            pl.BlockSpec((RBF, HALF), lambda i: (i, 0)),
            pl.BlockSpec((RBF, HALF), lambda i: (i, 0)),
            pl.BlockSpec((RBF, 1), lambda i: (i, 0)),
            pl.BlockSpec((8, D_HID), lambda i: (0, 0)),
            pl.BlockSpec((D_HID, D2), lambda i: (0, 0)),
        ],
        out_specs=pl.BlockSpec((RBF, D2), lambda i: (i, 0)),
        out_shape=jax.ShapeDtypeStruct((N2, D2), f32),
    )(a0, a1, p0, p1, dinv, b1t, W2p)

    q0, q1 = _scat2_kernel(src_p, dst_p, p2, z2)

    out = pl.pallas_call(
        _fuse3_body,
        grid=(GRIDF,),
        in_specs=[
            pl.BlockSpec((RBF, D2), lambda i: (i, 0)),
            pl.BlockSpec((RBF, D2), lambda i: (i, 0)),
            pl.BlockSpec((RBF, D2), lambda i: (i, 0)),
            pl.BlockSpec((RBF, 1), lambda i: (i, 0)),
            pl.BlockSpec((8, D2), lambda i: (0, 0)),
        ],
        out_specs=pl.BlockSpec((RBF, NCLS), lambda i: (i, 0)),
        out_shape=jax.ShapeDtypeStruct((N, NCLS), f32),
    )(q0, q1, p2, dinv, b2t)

    return out


# speedup vs baseline: 44.3153x; 1.0002x over previous
import functools

import jax
import jax.numpy as jnp
from jax import lax
from jax.experimental import pallas as pl
from jax.experimental.pallas import tpu as pltpu
from jax.experimental.pallas import tpu_sc as plsc

N = 50000
E = 800000
D_IN = 768
D_HID = 64
NCLS = 7

NP = 48
N2 = N + NP
EP = 819200
G = EP // 128
HALF = 32
D2 = 16

NSC = 2
NT = 16
NW = NSC * NT

ROWS_T = N2 // NT

TG1 = G // NT
SCH1 = 80
NSC1 = TG1 // SCH1

TG2 = G // NW
SCH2 = 40
NSC2 = TG2 // SCH2

EPTD = EP // NW
NGD = EPTD // 16

NBUF1 = 8
NBUF2 = 8

RB1 = 2048
GRID1 = (N + RB1 - 1) // RB1
RBF = 4096
GRIDF = (N + RBF - 1) // RBF

_mesh = plsc.VectorSubcoreMesh(core_axis_name="c", subcore_axis_name="s")
_SC_PARAMS = pltpu.CompilerParams(needs_layout_passes=False,
                                  use_tc_tiling_on_sc=False)


@functools.partial(
    pl.kernel,
    out_type=jax.ShapeDtypeStruct((NW, N2), jnp.float32),
    mesh=_mesh,
    scratch_types=[
        pltpu.VMEM((EPTD,), jnp.int32),
        pltpu.VMEM((N2,), jnp.float32),
    ],
    compiler_params=_SC_PARAMS,
)
def _deg_kernel(dst_hbm, zeros_hbm, hist_hbm, dst_v, hist_v):
    cid = lax.axis_index("c")
    sid = lax.axis_index("s")
    wid = sid * NSC + cid
    pltpu.sync_copy(zeros_hbm, hist_v)
    pltpu.sync_copy(dst_hbm.at[pl.ds(wid * EPTD, EPTD)], dst_v)
    ones16 = jnp.ones((16,), jnp.float32)

    def body(i, carry):
        for u in range(4):
            idx = dst_v[pl.ds(i * 64 + u * 16, 16)]
            plsc.addupdate_scatter(hist_v, [idx], ones16)
        return carry

    lax.fori_loop(0, NGD // 4, body, 0)
    pltpu.sync_copy(hist_v, hist_hbm.at[wid])


@functools.partial(
    pl.kernel,
    out_type=(
        jax.ShapeDtypeStruct((N2, HALF), jnp.bfloat16),
        jax.ShapeDtypeStruct((N2, HALF), jnp.bfloat16),
    ),
    mesh=_mesh,
    scratch_types=[
        pltpu.VMEM((SCH1, 128), jnp.int32),
        pltpu.VMEM((SCH1, 128), jnp.int32),
        [pltpu.VMEM((128, HALF), jnp.bfloat16) for _ in range(NBUF1)],
        pltpu.VMEM_SHARED((N2, HALF), jnp.bfloat16),
        [pltpu.SemaphoreType.DMA for _ in range(NBUF1)],
    ],
    compiler_params=_SC_PARAMS,
)
def _scat1_kernel(src_hbm, dst_hbm, p0_hbm, p1_hbm, z_hbm,
                  o0_hbm, o1_hbm, src_v, dst_v, rows, acc_s, gsem):
    cid = lax.axis_index("c")
    sid = lax.axis_index("s")
    r0 = sid * ROWS_T
    pltpu.sync_copy(z_hbm, acc_s.at[pl.ds(r0, ROWS_T)])
    plsc.subcore_barrier()

    def pipe(p_hbm):
        def superchunk(t, carry):
            g0 = sid * TG1 + t * SCH1
            pltpu.sync_copy(src_hbm.at[pl.ds(g0, SCH1)], src_v)
            pltpu.sync_copy(dst_hbm.at[pl.ds(g0, SCH1)], dst_v)
            for j in range(NBUF1):
                pltpu.async_copy(p_hbm.at[src_v.at[j]], rows[j], gsem[j])
            for j in range(SCH1):
                b = j % NBUF1
                pltpu.make_async_copy(p_hbm.at[src_v.at[j]],
                                      rows[b], gsem[b]).wait()
                pltpu.sync_copy(rows[b], acc_s.at[dst_v.at[j]], add=True)
                if j + NBUF1 < SCH1:
                    pltpu.async_copy(p_hbm.at[src_v.at[j + NBUF1]],
                                     rows[b], gsem[b])
            return carry
        return superchunk

    @pl.when(cid == 0)
    def _():
        lax.fori_loop(0, NSC1, pipe(p0_hbm), 0)

    @pl.when(cid == 1)
    def _():
        lax.fori_loop(0, NSC1, pipe(p1_hbm), 0)

    plsc.subcore_barrier()

    @pl.when(cid == 0)
    def _():
        pltpu.sync_copy(acc_s.at[pl.ds(r0, ROWS_T)], o0_hbm.at[pl.ds(r0, ROWS_T)])

    @pl.when(cid == 1)
    def _():
        pltpu.sync_copy(acc_s.at[pl.ds(r0, ROWS_T)], o1_hbm.at[pl.ds(r0, ROWS_T)])


@functools.partial(
    pl.kernel,
    out_type=(
        jax.ShapeDtypeStruct((N2, D2), jnp.float32),
        jax.ShapeDtypeStruct((N2, D2), jnp.float32),
    ),
    mesh=_mesh,
    scratch_types=[
        pltpu.VMEM((SCH2, 128), jnp.int32),
        pltpu.VMEM((SCH2, 128), jnp.int32),
        [pltpu.VMEM((128, D2), jnp.float32) for _ in range(NBUF2)],
        pltpu.VMEM_SHARED((N2, D2), jnp.float32),
        [pltpu.SemaphoreType.DMA for _ in range(NBUF2)],
    ],
    compiler_params=_SC_PARAMS,
)
def _scat2_kernel(src_hbm, dst_hbm, p2_hbm, z_hbm,
                  q0_hbm, q1_hbm, src_v, dst_v, rows, acc_s, gsem):
    cid = lax.axis_index("c")
    sid = lax.axis_index("s")
    wid = sid * NSC + cid
    r0 = sid * ROWS_T
    pltpu.sync_copy(z_hbm, acc_s.at[pl.ds(r0, ROWS_T)])
    plsc.subcore_barrier()

    def superchunk(t, carry):
        g0 = wid * TG2 + t * SCH2
        pltpu.sync_copy(src_hbm.at[pl.ds(g0, SCH2)], src_v)
        pltpu.sync_copy(dst_hbm.at[pl.ds(g0, SCH2)], dst_v)
        for j in range(NBUF2):
            pltpu.async_copy(p2_hbm.at[src_v.at[j]], rows[j], gsem[j])
        for j in range(SCH2):
            b = j % NBUF2
            pltpu.make_async_copy(p2_hbm.at[src_v.at[j]],
                                  rows[b], gsem[b]).wait()
            pltpu.sync_copy(rows[b], acc_s.at[dst_v.at[j]], add=True)
            if j + NBUF2 < SCH2:
                pltpu.async_copy(p2_hbm.at[src_v.at[j + NBUF2]],
                                 rows[b], gsem[b])
        return carry

    lax.fori_loop(0, NSC2, superchunk, 0)
    plsc.subcore_barrier()

    @pl.when(cid == 0)
    def _():
        pltpu.sync_copy(acc_s.at[pl.ds(r0, ROWS_T)], q0_hbm.at[pl.ds(r0, ROWS_T)])

    @pl.when(cid == 1)
    def _():
        pltpu.sync_copy(acc_s.at[pl.ds(r0, ROWS_T)], q1_hbm.at[pl.ds(r0, ROWS_T)])


def _mmscale_body(x_ref, w_ref, hist_ref, p0_ref, p1_ref, dinv_ref):
    h = jnp.dot(x_ref[...], w_ref[...], preferred_element_type=jnp.float32)
    deg = 1.0 + jnp.sum(hist_ref[...], axis=0)
    dinv = lax.rsqrt(deg)[:, None]
    p = h * dinv
    p0_ref[...] = p[:, :HALF].astype(jnp.bfloat16)
    p1_ref[...] = p[:, HALF:].astype(jnp.bfloat16)
    dinv_ref[...] = dinv


def _fuse2_body(a0_ref, a1_ref, p0_ref, p1_ref, dinv_ref, b1_ref, w2_ref,
                p2_ref):
    f32 = jnp.float32
    dinv = dinv_ref[...]
    x0 = jnp.maximum((a0_ref[...].astype(f32) + p0_ref[...].astype(f32))
                     * dinv + b1_ref[0:1, :HALF], 0.0)
    x1 = jnp.maximum((a1_ref[...].astype(f32) + p1_ref[...].astype(f32))
                     * dinv + b1_ref[0:1, HALF:], 0.0)
    h2 = (jnp.dot(x0, w2_ref[:HALF, :], preferred_element_type=f32)
          + jnp.dot(x1, w2_ref[HALF:, :], preferred_element_type=f32))
    p2_ref[...] = h2 * dinv


def _fuse3_body(q0_ref, q1_ref, p2_ref, dinv_ref, b2_ref, o_ref):
    logits = ((q0_ref[...] + q1_ref[...] + p2_ref[...]) * dinv_ref[...]
              + b2_ref[0:1, :])
    col = lax.broadcasted_iota(jnp.int32, (RBF, D2), 1)
    masked = jnp.where(col < NCLS, logits, -1e30)
    m = jnp.max(masked, axis=1, keepdims=True)
    s = jnp.sum(jnp.exp(masked - m), axis=1, keepdims=True)
    o_ref[...] = (logits - m - jnp.log(s))[:, :NCLS]


def kernel(x_text_feat, edge_index, W1, b1, W2, b2):
    f32 = jnp.float32
    src = edge_index[0].astype(jnp.int32).reshape(E // 128, 128)
    dst = edge_index[1].astype(jnp.int32).reshape(E // 128, 128)
    pad2d = (N + (jnp.arange((EP - E) // 128 * 128, dtype=jnp.int32) % NP)
             ).reshape((EP - E) // 128, 128)
    src_p = jnp.concatenate([src, pad2d], axis=0)
    dst_p = jnp.concatenate([dst, pad2d], axis=0)
    dst_flat = dst_p.reshape(EP)

    zN = jnp.zeros((N2,), f32)
    z1 = jnp.zeros((ROWS_T, HALF), jnp.bfloat16)
    z2 = jnp.zeros((ROWS_T, D2), f32)
    b1t = jnp.tile(b1.astype(f32).reshape(1, D_HID), (8, 1))
    W2p = jnp.zeros((D_HID, D2), f32).at[:, :NCLS].set(W2.astype(f32))
    b2t = jnp.tile(jnp.pad(b2.astype(f32), (0, D2 - NCLS)).reshape(1, D2),
                   (8, 1))

    hist = _deg_kernel(dst_flat, zN)

    p0, p1, dinv = pl.pallas_call(
        _mmscale_body,
        grid=(GRID1,),
        in_specs=[
            pl.BlockSpec((RB1, D_IN), lambda i: (i, 0)),
            pl.BlockSpec((D_IN, D_HID), lambda i: (0, 0)),
            pl.BlockSpec((NW, RB1), lambda i: (0, i)),
        ],
        out_specs=[
            pl.BlockSpec((RB1, HALF), lambda i: (i, 0)),
            pl.BlockSpec((RB1, HALF), lambda i: (i, 0)),
            pl.BlockSpec((RB1, 1), lambda i: (i, 0)),
        ],
        out_shape=[
            jax.ShapeDtypeStruct((N2, HALF), jnp.bfloat16),
            jax.ShapeDtypeStruct((N2, HALF), jnp.bfloat16),
            jax.ShapeDtypeStruct((N, 1), f32),
        ],
    )(x_text_feat, W1, hist)

    a0, a1 = _scat1_kernel(src_p, dst_p, p0, p1, z1)

    p2 = pl.pallas_call(
        _fuse2_body,
        grid=(GRIDF,),
        in_specs=[
            pl.BlockSpec((RBF, HALF), lambda i: (i, 0)),
            pl.BlockSpec((RBF, HALF), lambda i: (i, 0)),
            pl.BlockSpec((RBF, HALF), lambda i: (i, 0)),
            pl.BlockSpec((RBF, HALF), lambda i: (i, 0)),
            pl.BlockSpec((RBF, 1), lambda i: (i, 0)),
            pl.BlockSpec((8, D_HID), lambda i: (0, 0)),
            pl.BlockSpec((D_HID, D2), lambda i: (0, 0)),
        ],
        out_specs=pl.BlockSpec((RBF, D2), lambda i: (i, 0)),
        out_shape=jax.ShapeDtypeStruct((N2, D2), f32),
    )(a0, a1, p0, p1, dinv, b1t, W2p)

    q0, q1 = _scat2_kernel(src_p, dst_p, p2, z2)

    out = pl.pallas_call(
        _fuse3_body,
        grid=(GRIDF,),
        in_specs=[
            pl.BlockSpec((RBF, D2), lambda i: (i, 0)),
            pl.BlockSpec((RBF, D2), lambda i: (i, 0)),
            pl.BlockSpec((RBF, D2), lambda i: (i, 0)),
            pl.BlockSpec((RBF, 1), lambda i: (i, 0)),
            pl.BlockSpec((8, D2), lambda i: (0, 0)),
        ],
        out_specs=pl.BlockSpec((RBF, NCLS), lambda i: (i, 0)),
        out_shape=jax.ShapeDtypeStruct((N, NCLS), f32),
    )(q0, q1, p2, dinv, b2t)

    return out
